# baseline (speedup 1.0000x reference)
import functools
import jax
import jax.numpy as jnp
from jax import lax
from jax.experimental import pallas as pl
from jax.experimental.pallas import tpu as pltpu
from jax.experimental.pallas import tpu_sc as plsc

_E = 8
_K = 2
_D = 1024
_F = 1024
_T = 2048
_COEFF = 0.01
_M = 128
_R = _T * _K + _E * _M
_NTILES = _R // _M
_NW = 32


def _route_body(x_ref, gw_ref, pos0_ref, pos1_ref, w0_ref, w1_ref,
                eid_ref, pnx_ref, seg_ref, aux_ref):
    x = x_ref[...]
    gw = gw_ref[...]
    logits = lax.dot_general(gw, x, (((1,), (1,)), ((), ())),
                             preferred_element_type=jnp.float32)
    m = jnp.max(logits, axis=0, keepdims=True)
    p = jnp.exp(logits - m)
    scores = p / jnp.sum(p, axis=0, keepdims=True)

    eiota = lax.broadcasted_iota(jnp.int32, (_E, _T), 0).astype(jnp.float32)
    m0 = jnp.max(scores, axis=0, keepdims=True)
    i0 = jnp.min(jnp.where(scores == m0, eiota, float(_E)), axis=0,
                 keepdims=True)
    oh0 = (eiota == i0).astype(jnp.float32)
    masked = jnp.where(oh0 > 0, -1.0, scores)
    m1 = jnp.max(masked, axis=0, keepdims=True)
    i1 = jnp.min(jnp.where(masked == m1, eiota, float(_E)), axis=0,
                 keepdims=True)
    oh1 = (eiota == i1).astype(jnp.float32)

    s0 = jnp.sum(scores * oh0, axis=0, keepdims=True)
    s1 = jnp.sum(scores * oh1, axis=0, keepdims=True)
    denom = s0 + s1 + 1e-9
    w0_ref[...] = s0 / denom
    w1_ref[...] = s1 / denom

    c = oh0 + oh1
    nchunk = _T // _M
    liota = lax.broadcasted_iota(jnp.int32, (_M, _M), 0)
    ciota = lax.broadcasted_iota(jnp.int32, (_M, _M), 1)
    triu_strict = (liota < ciota).astype(jnp.float32)
    off = jnp.zeros((_E, 1), jnp.float32)
    s_parts = []
    for b in range(nchunk):
        cb = c[:, b * _M:(b + 1) * _M]
        sb = lax.dot_general(cb, triu_strict, (((1,), (0,)), ((), ())),
                             preferred_element_type=jnp.float32)
        s_parts.append(sb + off)
        off = off + jnp.sum(cb, axis=1, keepdims=True)
    s_excl = jnp.concatenate(s_parts, axis=1)
    counts = off

    counts_i = counts.astype(jnp.int32)
    pad = (((counts_i + (_M - 1)) // _M) * _M).astype(jnp.float32)
    e1 = lax.broadcasted_iota(jnp.int32, (_E, _E), 0)
    e2 = lax.broadcasted_iota(jnp.int32, (_E, _E), 1)
    triu8 = (e1 < e2).astype(jnp.float32)
    seg_off = lax.dot_general(pad, triu8, (((0,), (0,)), ((), ())),
                              preferred_element_type=jnp.float32)
    seg_off = seg_off.reshape(_E, 1)
    ends = seg_off + pad

    base0 = jnp.sum(seg_off * oh0, axis=0, keepdims=True)
    base1 = jnp.sum(seg_off * oh1, axis=0, keepdims=True)
    r0 = jnp.sum(s_excl * oh0, axis=0, keepdims=True)
    r1 = jnp.sum(s_excl * oh1, axis=0, keepdims=True)
    pos0_ref[...] = (base0 + r0).astype(jnp.int32)
    pos1_ref[...] = (base1 + r1).astype(jnp.int32)

    tstart = (lax.broadcasted_iota(jnp.int32, (_E, _NTILES), 1)
              .astype(jnp.float32) * float(_M))
    eid = jnp.sum((tstart >= ends).astype(jnp.float32), axis=0, keepdims=True)
    eid = jnp.minimum(eid, float(_E - 1))
    eid_ref[...] = eid.astype(jnp.int32)

    ee1 = e1.astype(jnp.float32)
    ee2 = e2.astype(jnp.float32)
    pad_cols = lax.dot_general(jnp.ones((_E, 1), jnp.float32), pad,
                               (((1,), (1,)), ((), ())),
                               preferred_element_type=jnp.float32)
    cand = jnp.where((ee2 > ee1) * (pad_cols > 0).astype(jnp.float32) > 0,
                     ee2, 1e9)
    nxt = jnp.min(cand, axis=1, keepdims=True)
    nxt = jnp.where(nxt > float(_E), ee1[:, :1], nxt)
    eiota_nt = (lax.broadcasted_iota(jnp.int32, (_E, _NTILES), 0)
                .astype(jnp.float32))
    sel = (eiota_nt == eid).astype(jnp.float32)
    pnx_ref[...] = jnp.sum(sel * nxt, axis=0, keepdims=True).astype(jnp.int32)

    pres_before = jnp.sum(
        jnp.where((ee2 < ee1) * (pad_cols > 0).astype(jnp.float32) > 0,
                  1.0, 0.0), axis=1, keepdims=True)
    seg_ref[...] = jnp.sum(sel * pres_before, axis=0,
                           keepdims=True).astype(jnp.int32)

    f = counts / (float(_T * _K) + 1e-9)
    pmean = jnp.mean(scores, axis=1, keepdims=True)
    aux_ref[...] = jnp.sum(f * pmean, axis=0, keepdims=True) * (_COEFF * _E)


def _route(x2d, gate_w):
    return pl.pallas_call(
        _route_body,
        out_shape=(
            jax.ShapeDtypeStruct((1, _T), jnp.int32),
            jax.ShapeDtypeStruct((1, _T), jnp.int32),
            jax.ShapeDtypeStruct((1, _T), jnp.float32),
            jax.ShapeDtypeStruct((1, _T), jnp.float32),
            jax.ShapeDtypeStruct((1, _NTILES), jnp.int32),
            jax.ShapeDtypeStruct((1, _NTILES), jnp.int32),
            jax.ShapeDtypeStruct((1, _NTILES), jnp.int32),
            jax.ShapeDtypeStruct((1, 1), jnp.float32),
        ),
    )(x2d, gate_w)


def _build_dispatch(pos0, pos1, w0, w1):
    mesh = plsc.VectorSubcoreMesh(core_axis_name="c", subcore_axis_name="s")

    @functools.partial(
        pl.kernel, mesh=mesh,
        compiler_params=pltpu.CompilerParams(needs_layout_passes=False),
        out_type=(
            jax.ShapeDtypeStruct((_R,), jnp.int32),
            jax.ShapeDtypeStruct((_R,), jnp.float32),
        ),
        scratch_types=[
            pltpu.VMEM((_T,), jnp.int32),
            pltpu.VMEM((_T,), jnp.int32),
            pltpu.VMEM((_T,), jnp.float32),
            pltpu.VMEM((_T,), jnp.float32),
            pltpu.VMEM((_R,), jnp.int32),
            pltpu.VMEM((_R,), jnp.float32),
        ],
    )
    def k(p0_hbm, p1_hbm, w0_hbm, w1_hbm, idx_hbm, wrow_hbm,
          p0_v, p1_v, w0_v, w1_v, idx_v, wrow_v):
        wid = lax.axis_index("s") * 2 + lax.axis_index("c")

        @pl.when(wid == 0)
        def _():
            pltpu.sync_copy(p0_hbm, p0_v)
            pltpu.sync_copy(p1_hbm, p1_v)
            pltpu.sync_copy(w0_hbm, w0_v)
            pltpu.sync_copy(w1_hbm, w1_v)

            def zero(i, _):
                idx_v[pl.ds(i * 16, 16)] = jnp.zeros((16,), jnp.int32)
                wrow_v[pl.ds(i * 16, 16)] = jnp.zeros((16,), jnp.float32)
                return 0
            lax.fori_loop(0, _R // 16, zero, 0)

            def scat(i, _):
                tok = lax.iota(jnp.int32, 16) + i * 16
                sl = pl.ds(i * 16, 16)
                plsc.store_scatter(idx_v, [p0_v[sl]], tok)
                plsc.store_scatter(idx_v, [p1_v[sl]], tok)
                plsc.store_scatter(wrow_v, [p0_v[sl]], w0_v[sl])
                plsc.store_scatter(wrow_v, [p1_v[sl]], w1_v[sl])
                return 0
            lax.fori_loop(0, _T // 16, scat, 0)

            pltpu.sync_copy(idx_v, idx_hbm)
            pltpu.sync_copy(wrow_v, wrow_hbm)

    return k(pos0, pos1, w0, w1)


def _gather_rows(x2d, idx):
    mesh = plsc.VectorSubcoreMesh(core_axis_name="c", subcore_axis_name="s")
    rows_per_w = _R // _NW
    chunk = 32
    nch = rows_per_w // chunk

    nbuf = 3

    @functools.partial(
        pl.kernel, mesh=mesh,
        compiler_params=pltpu.CompilerParams(needs_layout_passes=False),
        out_type=jax.ShapeDtypeStruct((_R, _D), jnp.float32),
        scratch_types=[
            pltpu.VMEM((rows_per_w,), jnp.int32),
            *([pltpu.VMEM((chunk, _D), jnp.float32)] * nbuf),
            *([pltpu.SemaphoreType.DMA] * nbuf),
            *([pltpu.SemaphoreType.DMA] * nbuf),
        ],
    )
    def k(x_hbm, idx_hbm, xs_hbm, idx_v, *scr):
        bufs = scr[:nbuf]
        gsem = scr[nbuf:2 * nbuf]
        ssem = scr[2 * nbuf:]
        wid = lax.axis_index("s") * 2 + lax.axis_index("c")
        base = wid * rows_per_w
        pltpu.sync_copy(idx_hbm.at[pl.ds(base, rows_per_w)], idx_v)

        def start_gather(i, b):
            return pltpu.async_copy(
                x_hbm.at[idx_v.at[pl.ds(i * chunk, chunk)]], bufs[b], gsem[b])

        gh = [None] * nbuf
        sh = [None] * nbuf
        waited = [True] * nbuf
        for i in range(min(nbuf, nch)):
            gh[i] = start_gather(i, i)
        for i in range(nch):
            b = i % nbuf
            gh[b].wait()
            sh[b] = pltpu.async_copy(
                bufs[b], xs_hbm.at[pl.ds(base + i * chunk, chunk)], ssem[b])
            waited[b] = False
            if i + nbuf < nch:
                sh[b].wait()
                waited[b] = True
                gh[b] = start_gather(i + nbuf, b)
        for b in range(nbuf):
            if not waited[b]:
                sh[b].wait()

    return k(x2d, idx)


def _ffn_gather_body(eid_ref, pnx_ref, seg_ref, idx_ref, x_ref,
                     wg_ref, wu_ref, wd_ref, wrow_ref, out_ref,
                     xbuf, wgs, wus, wds, gsem, usem, dsem):
    i = pl.program_id(0)
    e = eid_ref[i]
    slot = lax.rem(seg_ref[i], 2)

    def issue(g, s):
        pltpu.make_async_copy(wg_ref.at[g], wgs.at[s], gsem.at[s]).start()
        pltpu.make_async_copy(wu_ref.at[g], wus.at[s], usem.at[s]).start()
        pltpu.make_async_copy(wd_ref.at[g], wds.at[s], dsem.at[s]).start()

    def wait_for(g, s):
        pltpu.make_async_copy(wg_ref.at[g], wgs.at[s], gsem.at[s]).wait()
        pltpu.make_async_copy(wu_ref.at[g], wus.at[s], usem.at[s]).wait()
        pltpu.make_async_copy(wd_ref.at[g], wds.at[s], dsem.at[s]).wait()

    @pl.when(i == 0)
    def _():
        issue(e, slot)

    boundary = (i == 0) | (eid_ref[jnp.maximum(i - 1, 0)] != e)

    @pl.when(boundary)
    def _():
        wait_for(e, slot)
        nx = pnx_ref[i]
        @pl.when(nx != e)
        def _():
            issue(nx, 1 - slot)

    def gather_into(tile, bslot):
        for r in range(_M):
            t = idx_ref[tile * _M + r]
            xbuf[bslot, pl.ds(r, 1), :] = x_ref[pl.ds(t, 1), :]

    par = lax.rem(i, 2)

    @pl.when(i == 0)
    def _():
        gather_into(i, par)

    @pl.when(i + 1 < pl.num_programs(0))
    def _():
        gather_into(i + 1, 1 - par)

    xt = xbuf[par]
    g = lax.dot_general(xt, wgs[slot], (((1,), (1,)), ((), ())),
                        preferred_element_type=jnp.float32)
    u = lax.dot_general(xt, wus[slot], (((1,), (1,)), ((), ())),
                        preferred_element_type=jnp.float32)
    h1 = (g * jax.nn.sigmoid(g)) * u
    o = lax.dot_general(h1, wds[slot], (((1,), (1,)), ((), ())),
                        preferred_element_type=jnp.float32)
    out_ref[...] = o * wrow_ref[...]


def _ffn_gather(x2d, idx, wg, wu, wd, wrow, eid, pnx, seg, ntiles):
    grid_spec = pltpu.PrefetchScalarGridSpec(
        num_scalar_prefetch=4,
        grid=(ntiles,),
        in_specs=[
            pl.BlockSpec((_T, _D), lambda i, e, p, s, ix: (0, 0)),
            pl.BlockSpec(memory_space=pl.ANY),
            pl.BlockSpec(memory_space=pl.ANY),
            pl.BlockSpec(memory_space=pl.ANY),
            pl.BlockSpec((_M, 1), lambda i, e, p, s, ix: (i, 0)),
        ],
        out_specs=pl.BlockSpec((_M, _D), lambda i, e, p, s, ix: (i, 0)),
        scratch_shapes=[
            pltpu.VMEM((2, _M, _D), jnp.float32),
            pltpu.VMEM((2, _F, _D), jnp.float32),
            pltpu.VMEM((2, _F, _D), jnp.float32),
            pltpu.VMEM((2, _D, _F), jnp.float32),
            pltpu.SemaphoreType.DMA((2,)),
            pltpu.SemaphoreType.DMA((2,)),
            pltpu.SemaphoreType.DMA((2,)),
        ],
    )
    return pl.pallas_call(
        _ffn_gather_body,
        grid_spec=grid_spec,
        out_shape=jax.ShapeDtypeStruct((ntiles * _M, _D), jnp.float32),
    )(eid, pnx, seg, idx, x2d, wg, wu, wd, wrow)


def _ffn_body(eid_ref, xs_ref, wg_ref, wu_ref, wd_ref, wrow_ref, out_ref):
    xt = xs_ref[...]
    g = lax.dot_general(xt, wg_ref[0], (((1,), (1,)), ((), ())),
                        preferred_element_type=jnp.float32)
    u = lax.dot_general(xt, wu_ref[0], (((1,), (1,)), ((), ())),
                        preferred_element_type=jnp.float32)
    h1 = (g * jax.nn.sigmoid(g)) * u
    o = lax.dot_general(h1, wd_ref[0], (((1,), (1,)), ((), ())),
                        preferred_element_type=jnp.float32)
    out_ref[...] = o * wrow_ref[...]


def _ffn(xs, wg, wu, wd, wrow, eid, ntiles):
    grid_spec = pltpu.PrefetchScalarGridSpec(
        num_scalar_prefetch=1,
        grid=(ntiles,),
        in_specs=[
            pl.BlockSpec((_M, _D), lambda i, eid_ref: (i, 0)),
            pl.BlockSpec((1, _F, _D), lambda i, eid_ref: (eid_ref[i], 0, 0)),
            pl.BlockSpec((1, _F, _D), lambda i, eid_ref: (eid_ref[i], 0, 0)),
            pl.BlockSpec((1, _D, _F), lambda i, eid_ref: (eid_ref[i], 0, 0)),
            pl.BlockSpec((_M, 1), lambda i, eid_ref: (i, 0)),
        ],
        out_specs=pl.BlockSpec((_M, _D), lambda i, eid_ref: (i, 0)),
    )
    return pl.pallas_call(
        _ffn_body,
        grid_spec=grid_spec,
        out_shape=jax.ShapeDtypeStruct((ntiles * _M, _D), jnp.float32),
    )(eid, xs, wg, wu, wd, wrow)


def _combine(ys_r, ys_s, pos0, pos1):
    mesh = plsc.VectorSubcoreMesh(core_axis_name="c", subcore_axis_name="s")
    tok_per_w = _T // _NW
    chunk = 16
    nch = tok_per_w // chunk

    nbuf = 2

    @functools.partial(
        pl.kernel, mesh=mesh,
        compiler_params=pltpu.CompilerParams(needs_layout_passes=False),
        out_type=jax.ShapeDtypeStruct((_T, _D), jnp.float32),
        scratch_types=[
            pltpu.VMEM((tok_per_w,), jnp.int32),
            pltpu.VMEM((tok_per_w,), jnp.int32),
            *([pltpu.VMEM((chunk, _D), jnp.float32)] * (3 * nbuf)),
            *([pltpu.SemaphoreType.DMA] * (3 * nbuf)),
            *([pltpu.SemaphoreType.DMA] * nbuf),
        ],
    )
    def k(ysr_hbm, yss_hbm, p0_hbm, p1_hbm, out_hbm, i0_v, i1_v, *scr):
        r0 = scr[0:nbuf]
        r1 = scr[nbuf:2 * nbuf]
        rs = scr[2 * nbuf:3 * nbuf]
        gsem = scr[3 * nbuf:6 * nbuf]
        ssem = scr[6 * nbuf:]
        wid = lax.axis_index("s") * 2 + lax.axis_index("c")
        base = wid * tok_per_w
        pltpu.sync_copy(p0_hbm.at[pl.ds(base, tok_per_w)], i0_v)
        pltpu.sync_copy(p1_hbm.at[pl.ds(base, tok_per_w)], i1_v)

        def start_loads(i, b):
            sl = pl.ds(i * chunk, chunk)
            return (
                pltpu.async_copy(ysr_hbm.at[i0_v.at[sl]], r0[b], gsem[b]),
                pltpu.async_copy(ysr_hbm.at[i1_v.at[sl]], r1[b], gsem[nbuf + b]),
                pltpu.async_copy(yss_hbm.at[pl.ds(base + i * chunk, chunk)],
                                 rs[b], gsem[2 * nbuf + b]),
            )

        gh = [None] * nbuf
        sh = [None] * nbuf
        waited = [True] * nbuf
        for i in range(min(nbuf, nch)):
            gh[i] = start_loads(i, i)
        for i in range(nch):
            b = i % nbuf
            for h in gh[b]:
                h.wait()
            for r in range(chunk):
                def add(cc, _):
                    sl = pl.ds(cc * 16, 16)
                    rs[b][r, sl] = rs[b][r, sl] + r0[b][r, sl] + r1[b][r, sl]
                    return 0
                lax.fori_loop(0, _D // 16, add, 0, unroll=8)
            sh[b] = pltpu.async_copy(
                rs[b], out_hbm.at[pl.ds(base + i * chunk, chunk)], ssem[b])
            waited[b] = False
            if i + nbuf < nch:
                sh[b].wait()
                waited[b] = True
                gh[b] = start_loads(i + nbuf, b)
        for b in range(nbuf):
            if not waited[b]:
                sh[b].wait()

    return k(ys_r, ys_s, pos0, pos1)


def kernel(x, shared_gate, shared_up, shared_down,
           routed_gate, routed_up, routed_down, gate_w):
    Bx, Sx, Dx = x.shape
    x2d = x.reshape(_T, _D)

    pos0, pos1, w0, w1, eid, pnx, seg, aux = _route(x2d, gate_w)
    pos0 = pos0.reshape(_T)
    pos1 = pos1.reshape(_T)
    w0 = w0.reshape(_T)
    w1 = w1.reshape(_T)
    eid = eid.reshape(_NTILES)
    pnx = pnx.reshape(_NTILES)
    seg = seg.reshape(_NTILES)

    ones = jnp.ones((_T, 1), jnp.float32)
    eid_s = jnp.zeros((_T // _M,), jnp.int32)
    ys_s = _ffn(x2d, shared_gate, shared_up, shared_down,
                ones, eid_s, _T // _M)

    idx, wrow = _build_dispatch(pos0, pos1, w0, w1)

    ys_r = _ffn_gather(x2d, idx, routed_gate, routed_up, routed_down,
                       wrow.reshape(_R, 1), eid, pnx, seg, _NTILES)

    out2d = _combine(ys_r, ys_s, pos0, pos1)
    return out2d.reshape(Bx, Sx, Dx), aux[0, 0]


# speedup vs baseline: 1.0234x; 1.0234x over previous
import functools
import jax
import jax.numpy as jnp
from jax import lax
from jax.experimental import pallas as pl
from jax.experimental.pallas import tpu as pltpu
from jax.experimental.pallas import tpu_sc as plsc

_E = 8
_K = 2
_D = 1024
_F = 1024
_T = 2048
_COEFF = 0.01
_M = 128
_R = _T * _K + _E * _M
_NTILES = _R // _M
_NW = 32


def _route_body(x_ref, gw_ref, pos0_ref, pos1_ref, w0_ref, w1_ref,
                eid_ref, pnx_ref, seg_ref, aux_ref):
    x = x_ref[...]
    gw = gw_ref[...]
    logits = lax.dot_general(gw, x, (((1,), (1,)), ((), ())),
                             preferred_element_type=jnp.float32)
    m = jnp.max(logits, axis=0, keepdims=True)
    p = jnp.exp(logits - m)
    scores = p / jnp.sum(p, axis=0, keepdims=True)

    eiota = lax.broadcasted_iota(jnp.int32, (_E, _T), 0).astype(jnp.float32)
    m0 = jnp.max(scores, axis=0, keepdims=True)
    i0 = jnp.min(jnp.where(scores == m0, eiota, float(_E)), axis=0,
                 keepdims=True)
    oh0 = (eiota == i0).astype(jnp.float32)
    masked = jnp.where(oh0 > 0, -1.0, scores)
    m1 = jnp.max(masked, axis=0, keepdims=True)
    i1 = jnp.min(jnp.where(masked == m1, eiota, float(_E)), axis=0,
                 keepdims=True)
    oh1 = (eiota == i1).astype(jnp.float32)

    s0 = jnp.sum(scores * oh0, axis=0, keepdims=True)
    s1 = jnp.sum(scores * oh1, axis=0, keepdims=True)
    denom = s0 + s1 + 1e-9
    w0_ref[...] = s0 / denom
    w1_ref[...] = s1 / denom

    c = oh0 + oh1
    nchunk = _T // _M
    liota = lax.broadcasted_iota(jnp.int32, (_M, _M), 0)
    ciota = lax.broadcasted_iota(jnp.int32, (_M, _M), 1)
    triu_strict = (liota < ciota).astype(jnp.float32)
    off = jnp.zeros((_E, 1), jnp.float32)
    s_parts = []
    for b in range(nchunk):
        cb = c[:, b * _M:(b + 1) * _M]
        sb = lax.dot_general(cb, triu_strict, (((1,), (0,)), ((), ())),
                             preferred_element_type=jnp.float32)
        s_parts.append(sb + off)
        off = off + jnp.sum(cb, axis=1, keepdims=True)
    s_excl = jnp.concatenate(s_parts, axis=1)
    counts = off

    counts_i = counts.astype(jnp.int32)
    pad = (((counts_i + (_M - 1)) // _M) * _M).astype(jnp.float32)
    e1 = lax.broadcasted_iota(jnp.int32, (_E, _E), 0)
    e2 = lax.broadcasted_iota(jnp.int32, (_E, _E), 1)
    triu8 = (e1 < e2).astype(jnp.float32)
    seg_off = lax.dot_general(pad, triu8, (((0,), (0,)), ((), ())),
                              preferred_element_type=jnp.float32)
    seg_off = seg_off.reshape(_E, 1)
    ends = seg_off + pad

    base0 = jnp.sum(seg_off * oh0, axis=0, keepdims=True)
    base1 = jnp.sum(seg_off * oh1, axis=0, keepdims=True)
    r0 = jnp.sum(s_excl * oh0, axis=0, keepdims=True)
    r1 = jnp.sum(s_excl * oh1, axis=0, keepdims=True)
    pos0_ref[...] = (base0 + r0).astype(jnp.int32)
    pos1_ref[...] = (base1 + r1).astype(jnp.int32)

    tstart = (lax.broadcasted_iota(jnp.int32, (_E, _NTILES), 1)
              .astype(jnp.float32) * float(_M))
    eid = jnp.sum((tstart >= ends).astype(jnp.float32), axis=0, keepdims=True)
    eid = jnp.minimum(eid, float(_E - 1))
    eid_ref[...] = eid.astype(jnp.int32)

    ee1 = e1.astype(jnp.float32)
    ee2 = e2.astype(jnp.float32)
    pad_cols = lax.dot_general(jnp.ones((_E, 1), jnp.float32), pad,
                               (((1,), (1,)), ((), ())),
                               preferred_element_type=jnp.float32)
    cand = jnp.where((ee2 > ee1) * (pad_cols > 0).astype(jnp.float32) > 0,
                     ee2, 1e9)
    nxt = jnp.min(cand, axis=1, keepdims=True)
    nxt = jnp.where(nxt > float(_E), ee1[:, :1], nxt)
    eiota_nt = (lax.broadcasted_iota(jnp.int32, (_E, _NTILES), 0)
                .astype(jnp.float32))
    sel = (eiota_nt == eid).astype(jnp.float32)
    pnx_ref[...] = jnp.sum(sel * nxt, axis=0, keepdims=True).astype(jnp.int32)

    pres_before = jnp.sum(
        jnp.where((ee2 < ee1) * (pad_cols > 0).astype(jnp.float32) > 0,
                  1.0, 0.0), axis=1, keepdims=True)
    seg_ref[...] = jnp.sum(sel * pres_before, axis=0,
                           keepdims=True).astype(jnp.int32)

    f = counts / (float(_T * _K) + 1e-9)
    pmean = jnp.mean(scores, axis=1, keepdims=True)
    aux_ref[...] = jnp.sum(f * pmean, axis=0, keepdims=True) * (_COEFF * _E)


def _route(x2d, gate_w):
    return pl.pallas_call(
        _route_body,
        out_shape=(
            jax.ShapeDtypeStruct((1, _T), jnp.int32),
            jax.ShapeDtypeStruct((1, _T), jnp.int32),
            jax.ShapeDtypeStruct((1, _T), jnp.float32),
            jax.ShapeDtypeStruct((1, _T), jnp.float32),
            jax.ShapeDtypeStruct((1, _NTILES), jnp.int32),
            jax.ShapeDtypeStruct((1, _NTILES), jnp.int32),
            jax.ShapeDtypeStruct((1, _NTILES), jnp.int32),
            jax.ShapeDtypeStruct((1, 1), jnp.float32),
        ),
    )(x2d, gate_w)


def _build_dispatch(pos0, pos1, w0, w1):
    mesh = plsc.VectorSubcoreMesh(core_axis_name="c", subcore_axis_name="s")

    @functools.partial(
        pl.kernel, mesh=mesh,
        compiler_params=pltpu.CompilerParams(needs_layout_passes=False),
        out_type=(
            jax.ShapeDtypeStruct((_R,), jnp.int32),
            jax.ShapeDtypeStruct((_R,), jnp.float32),
        ),
        scratch_types=[
            pltpu.VMEM((_T,), jnp.int32),
            pltpu.VMEM((_T,), jnp.int32),
            pltpu.VMEM((_T,), jnp.float32),
            pltpu.VMEM((_T,), jnp.float32),
            pltpu.VMEM((_R,), jnp.int32),
            pltpu.VMEM((_R,), jnp.float32),
        ],
    )
    def k(p0_hbm, p1_hbm, w0_hbm, w1_hbm, idx_hbm, wrow_hbm,
          p0_v, p1_v, w0_v, w1_v, idx_v, wrow_v):
        wid = lax.axis_index("s") * 2 + lax.axis_index("c")

        @pl.when(wid == 0)
        def _():
            pltpu.sync_copy(p0_hbm, p0_v)
            pltpu.sync_copy(p1_hbm, p1_v)
            pltpu.sync_copy(w0_hbm, w0_v)
            pltpu.sync_copy(w1_hbm, w1_v)

            def zero(i, _):
                idx_v[pl.ds(i * 16, 16)] = jnp.zeros((16,), jnp.int32)
                wrow_v[pl.ds(i * 16, 16)] = jnp.zeros((16,), jnp.float32)
                return 0
            lax.fori_loop(0, _R // 16, zero, 0)

            def scat(i, _):
                tok = lax.iota(jnp.int32, 16) + i * 16
                sl = pl.ds(i * 16, 16)
                plsc.store_scatter(idx_v, [p0_v[sl]], tok)
                plsc.store_scatter(idx_v, [p1_v[sl]], tok)
                plsc.store_scatter(wrow_v, [p0_v[sl]], w0_v[sl])
                plsc.store_scatter(wrow_v, [p1_v[sl]], w1_v[sl])
                return 0
            lax.fori_loop(0, _T // 16, scat, 0)

            pltpu.sync_copy(idx_v, idx_hbm)
            pltpu.sync_copy(wrow_v, wrow_hbm)

    return k(pos0, pos1, w0, w1)


def _gather_rows(x2d, idx):
    mesh = plsc.VectorSubcoreMesh(core_axis_name="c", subcore_axis_name="s")
    rows_per_w = _R // _NW
    chunk = 32
    nch = rows_per_w // chunk

    nbuf = 3

    @functools.partial(
        pl.kernel, mesh=mesh,
        compiler_params=pltpu.CompilerParams(needs_layout_passes=False),
        out_type=jax.ShapeDtypeStruct((_R, _D), jnp.float32),
        scratch_types=[
            pltpu.VMEM((rows_per_w,), jnp.int32),
            *([pltpu.VMEM((chunk, _D), jnp.float32)] * nbuf),
            *([pltpu.SemaphoreType.DMA] * nbuf),
            *([pltpu.SemaphoreType.DMA] * nbuf),
        ],
    )
    def k(x_hbm, idx_hbm, xs_hbm, idx_v, *scr):
        bufs = scr[:nbuf]
        gsem = scr[nbuf:2 * nbuf]
        ssem = scr[2 * nbuf:]
        wid = lax.axis_index("s") * 2 + lax.axis_index("c")
        base = wid * rows_per_w
        pltpu.sync_copy(idx_hbm.at[pl.ds(base, rows_per_w)], idx_v)

        def start_gather(i, b):
            return pltpu.async_copy(
                x_hbm.at[idx_v.at[pl.ds(i * chunk, chunk)]], bufs[b], gsem[b])

        gh = [None] * nbuf
        sh = [None] * nbuf
        waited = [True] * nbuf
        for i in range(min(nbuf, nch)):
            gh[i] = start_gather(i, i)
        for i in range(nch):
            b = i % nbuf
            gh[b].wait()
            sh[b] = pltpu.async_copy(
                bufs[b], xs_hbm.at[pl.ds(base + i * chunk, chunk)], ssem[b])
            waited[b] = False
            if i + nbuf < nch:
                sh[b].wait()
                waited[b] = True
                gh[b] = start_gather(i + nbuf, b)
        for b in range(nbuf):
            if not waited[b]:
                sh[b].wait()

    return k(x2d, idx)


def _ffn_gather_body(eid_ref, pnx_ref, seg_ref, idx_ref, x_ref,
                     wg_ref, wu_ref, wd_ref, wrow_ref, out_ref,
                     xbuf, wgs, wus, wds, gsem, usem, dsem):
    i = pl.program_id(0)
    e = eid_ref[i]
    slot = lax.rem(seg_ref[i], 2)

    def issue(g, s):
        pltpu.make_async_copy(wg_ref.at[g], wgs.at[s], gsem.at[s]).start()
        pltpu.make_async_copy(wu_ref.at[g], wus.at[s], usem.at[s]).start()
        pltpu.make_async_copy(wd_ref.at[g], wds.at[s], dsem.at[s]).start()

    def wait_for(g, s):
        pltpu.make_async_copy(wg_ref.at[g], wgs.at[s], gsem.at[s]).wait()
        pltpu.make_async_copy(wu_ref.at[g], wus.at[s], usem.at[s]).wait()
        pltpu.make_async_copy(wd_ref.at[g], wds.at[s], dsem.at[s]).wait()

    @pl.when(i == 0)
    def _():
        issue(e, slot)

    boundary = (i == 0) | (eid_ref[jnp.maximum(i - 1, 0)] != e)

    @pl.when(boundary)
    def _():
        wait_for(e, slot)
        nx = pnx_ref[i]
        @pl.when(nx != e)
        def _():
            issue(nx, 1 - slot)

    wr = wrow_ref[...]
    @pl.when(jnp.max(jnp.abs(wr)) > 0)
    def _():
        for r in range(_M):
            t = idx_ref[i * _M + r]
            xbuf[pl.ds(r, 1), :] = x_ref[pl.ds(t, 1), :]
        xt = xbuf[...]
        g = lax.dot_general(xt, wgs[slot], (((1,), (1,)), ((), ())),
                            preferred_element_type=jnp.float32)
        u = lax.dot_general(xt, wus[slot], (((1,), (1,)), ((), ())),
                            preferred_element_type=jnp.float32)
        h1 = (g * jax.nn.sigmoid(g)) * u
        o = lax.dot_general(h1, wds[slot], (((1,), (1,)), ((), ())),
                            preferred_element_type=jnp.float32)
        out_ref[...] = o * wr


def _ffn_gather(x2d, idx, wg, wu, wd, wrow, eid, pnx, seg, ntiles):
    grid_spec = pltpu.PrefetchScalarGridSpec(
        num_scalar_prefetch=4,
        grid=(ntiles,),
        in_specs=[
            pl.BlockSpec((_T, _D), lambda i, e, p, s, ix: (0, 0)),
            pl.BlockSpec(memory_space=pl.ANY),
            pl.BlockSpec(memory_space=pl.ANY),
            pl.BlockSpec(memory_space=pl.ANY),
            pl.BlockSpec((_M, 1), lambda i, e, p, s, ix: (i, 0)),
        ],
        out_specs=pl.BlockSpec((_M, _D), lambda i, e, p, s, ix: (i, 0)),
        scratch_shapes=[
            pltpu.VMEM((_M, _D), jnp.float32),
            pltpu.VMEM((2, _F, _D), jnp.float32),
            pltpu.VMEM((2, _F, _D), jnp.float32),
            pltpu.VMEM((2, _D, _F), jnp.float32),
            pltpu.SemaphoreType.DMA((2,)),
            pltpu.SemaphoreType.DMA((2,)),
            pltpu.SemaphoreType.DMA((2,)),
        ],
    )
    return pl.pallas_call(
        _ffn_gather_body,
        grid_spec=grid_spec,
        out_shape=jax.ShapeDtypeStruct((ntiles * _M, _D), jnp.float32),
    )(eid, pnx, seg, idx, x2d, wg, wu, wd, wrow)


def _ffn_body(eid_ref, xs_ref, wg_ref, wu_ref, wd_ref, wrow_ref, out_ref):
    xt = xs_ref[...]
    g = lax.dot_general(xt, wg_ref[0], (((1,), (1,)), ((), ())),
                        preferred_element_type=jnp.float32)
    u = lax.dot_general(xt, wu_ref[0], (((1,), (1,)), ((), ())),
                        preferred_element_type=jnp.float32)
    h1 = (g * jax.nn.sigmoid(g)) * u
    o = lax.dot_general(h1, wd_ref[0], (((1,), (1,)), ((), ())),
                        preferred_element_type=jnp.float32)
    out_ref[...] = o * wrow_ref[...]


def _ffn(xs, wg, wu, wd, wrow, eid, ntiles):
    grid_spec = pltpu.PrefetchScalarGridSpec(
        num_scalar_prefetch=1,
        grid=(ntiles,),
        in_specs=[
            pl.BlockSpec((_M, _D), lambda i, eid_ref: (i, 0)),
            pl.BlockSpec((1, _F, _D), lambda i, eid_ref: (eid_ref[i], 0, 0)),
            pl.BlockSpec((1, _F, _D), lambda i, eid_ref: (eid_ref[i], 0, 0)),
            pl.BlockSpec((1, _D, _F), lambda i, eid_ref: (eid_ref[i], 0, 0)),
            pl.BlockSpec((_M, 1), lambda i, eid_ref: (i, 0)),
        ],
        out_specs=pl.BlockSpec((_M, _D), lambda i, eid_ref: (i, 0)),
    )
    return pl.pallas_call(
        _ffn_body,
        grid_spec=grid_spec,
        out_shape=jax.ShapeDtypeStruct((ntiles * _M, _D), jnp.float32),
    )(eid, xs, wg, wu, wd, wrow)


def _combine(ys_r, ys_s, pos0, pos1):
    mesh = plsc.VectorSubcoreMesh(core_axis_name="c", subcore_axis_name="s")
    tok_per_w = _T // _NW
    chunk = 16
    nch = tok_per_w // chunk

    nbuf = 2

    @functools.partial(
        pl.kernel, mesh=mesh,
        compiler_params=pltpu.CompilerParams(needs_layout_passes=False),
        out_type=jax.ShapeDtypeStruct((_T, _D), jnp.float32),
        scratch_types=[
            pltpu.VMEM((tok_per_w,), jnp.int32),
            pltpu.VMEM((tok_per_w,), jnp.int32),
            *([pltpu.VMEM((chunk, _D), jnp.float32)] * (3 * nbuf)),
            *([pltpu.SemaphoreType.DMA] * (3 * nbuf)),
            *([pltpu.SemaphoreType.DMA] * nbuf),
        ],
    )
    def k(ysr_hbm, yss_hbm, p0_hbm, p1_hbm, out_hbm, i0_v, i1_v, *scr):
        r0 = scr[0:nbuf]
        r1 = scr[nbuf:2 * nbuf]
        rs = scr[2 * nbuf:3 * nbuf]
        gsem = scr[3 * nbuf:6 * nbuf]
        ssem = scr[6 * nbuf:]
        wid = lax.axis_index("s") * 2 + lax.axis_index("c")
        base = wid * tok_per_w
        pltpu.sync_copy(p0_hbm.at[pl.ds(base, tok_per_w)], i0_v)
        pltpu.sync_copy(p1_hbm.at[pl.ds(base, tok_per_w)], i1_v)

        def start_loads(i, b):
            sl = pl.ds(i * chunk, chunk)
            return (
                pltpu.async_copy(ysr_hbm.at[i0_v.at[sl]], r0[b], gsem[b]),
                pltpu.async_copy(ysr_hbm.at[i1_v.at[sl]], r1[b], gsem[nbuf + b]),
                pltpu.async_copy(yss_hbm.at[pl.ds(base + i * chunk, chunk)],
                                 rs[b], gsem[2 * nbuf + b]),
            )

        gh = [None] * nbuf
        sh = [None] * nbuf
        waited = [True] * nbuf
        for i in range(min(nbuf, nch)):
            gh[i] = start_loads(i, i)
        for i in range(nch):
            b = i % nbuf
            for h in gh[b]:
                h.wait()
            for r in range(chunk):
                def add(cc, _):
                    sl = pl.ds(cc * 16, 16)
                    rs[b][r, sl] = rs[b][r, sl] + r0[b][r, sl] + r1[b][r, sl]
                    return 0
                lax.fori_loop(0, _D // 16, add, 0, unroll=8)
            sh[b] = pltpu.async_copy(
                rs[b], out_hbm.at[pl.ds(base + i * chunk, chunk)], ssem[b])
            waited[b] = False
            if i + nbuf < nch:
                sh[b].wait()
                waited[b] = True
                gh[b] = start_loads(i + nbuf, b)
        for b in range(nbuf):
            if not waited[b]:
                sh[b].wait()

    return k(ys_r, ys_s, pos0, pos1)


def kernel(x, shared_gate, shared_up, shared_down,
           routed_gate, routed_up, routed_down, gate_w):
    Bx, Sx, Dx = x.shape
    x2d = x.reshape(_T, _D)

    pos0, pos1, w0, w1, eid, pnx, seg, aux = _route(x2d, gate_w)
    pos0 = pos0.reshape(_T)
    pos1 = pos1.reshape(_T)
    w0 = w0.reshape(_T)
    w1 = w1.reshape(_T)
    eid = eid.reshape(_NTILES)
    pnx = pnx.reshape(_NTILES)
    seg = seg.reshape(_NTILES)

    ones = jnp.ones((_T, 1), jnp.float32)
    eid_s = jnp.zeros((_T // _M,), jnp.int32)
    ys_s = _ffn(x2d, shared_gate, shared_up, shared_down,
                ones, eid_s, _T // _M)

    idx, wrow = _build_dispatch(pos0, pos1, w0, w1)

    ys_r = _ffn_gather(x2d, idx, routed_gate, routed_up, routed_down,
                       wrow.reshape(_R, 1), eid, pnx, seg, _NTILES)

    out2d = _combine(ys_r, ys_s, pos0, pos1)
    return out2d.reshape(Bx, Sx, Dx), aux[0, 0]


# speedup vs baseline: 1.0244x; 1.0009x over previous
import functools
import jax
import jax.numpy as jnp
from jax import lax
from jax.experimental import pallas as pl
from jax.experimental.pallas import tpu as pltpu
from jax.experimental.pallas import tpu_sc as plsc

_E = 8
_K = 2
_D = 1024
_F = 1024
_T = 2048
_COEFF = 0.01
_M = 128
_R = _T * _K + _E * _M
_NTILES = _R // _M
_NW = 32


def _route_body(x_ref, gw_ref, pos0_ref, pos1_ref, w0_ref, w1_ref,
                eid_ref, pnx_ref, seg_ref, aux_ref):
    x = x_ref[...]
    gw = gw_ref[...]
    logits = lax.dot_general(gw, x, (((1,), (1,)), ((), ())),
                             preferred_element_type=jnp.float32)
    m = jnp.max(logits, axis=0, keepdims=True)
    p = jnp.exp(logits - m)
    scores = p / jnp.sum(p, axis=0, keepdims=True)

    eiota = lax.broadcasted_iota(jnp.int32, (_E, _T), 0).astype(jnp.float32)
    m0 = jnp.max(scores, axis=0, keepdims=True)
    i0 = jnp.min(jnp.where(scores == m0, eiota, float(_E)), axis=0,
                 keepdims=True)
    oh0 = (eiota == i0).astype(jnp.float32)
    masked = jnp.where(oh0 > 0, -1.0, scores)
    m1 = jnp.max(masked, axis=0, keepdims=True)
    i1 = jnp.min(jnp.where(masked == m1, eiota, float(_E)), axis=0,
                 keepdims=True)
    oh1 = (eiota == i1).astype(jnp.float32)

    s0 = jnp.sum(scores * oh0, axis=0, keepdims=True)
    s1 = jnp.sum(scores * oh1, axis=0, keepdims=True)
    denom = s0 + s1 + 1e-9
    w0_ref[...] = s0 / denom
    w1_ref[...] = s1 / denom

    c = oh0 + oh1
    nchunk = _T // _M
    liota = lax.broadcasted_iota(jnp.int32, (_M, _M), 0)
    ciota = lax.broadcasted_iota(jnp.int32, (_M, _M), 1)
    triu_strict = (liota < ciota).astype(jnp.float32)
    off = jnp.zeros((_E, 1), jnp.float32)
    s_parts = []
    for b in range(nchunk):
        cb = c[:, b * _M:(b + 1) * _M]
        sb = lax.dot_general(cb, triu_strict, (((1,), (0,)), ((), ())),
                             preferred_element_type=jnp.float32)
        s_parts.append(sb + off)
        off = off + jnp.sum(cb, axis=1, keepdims=True)
    s_excl = jnp.concatenate(s_parts, axis=1)
    counts = off

    counts_i = counts.astype(jnp.int32)
    pad = (((counts_i + (_M - 1)) // _M) * _M).astype(jnp.float32)
    e1 = lax.broadcasted_iota(jnp.int32, (_E, _E), 0)
    e2 = lax.broadcasted_iota(jnp.int32, (_E, _E), 1)
    triu8 = (e1 < e2).astype(jnp.float32)
    seg_off = lax.dot_general(pad, triu8, (((0,), (0,)), ((), ())),
                              preferred_element_type=jnp.float32)
    seg_off = seg_off.reshape(_E, 1)
    ends = seg_off + pad

    base0 = jnp.sum(seg_off * oh0, axis=0, keepdims=True)
    base1 = jnp.sum(seg_off * oh1, axis=0, keepdims=True)
    r0 = jnp.sum(s_excl * oh0, axis=0, keepdims=True)
    r1 = jnp.sum(s_excl * oh1, axis=0, keepdims=True)
    pos0_ref[...] = (base0 + r0).astype(jnp.int32)
    pos1_ref[...] = (base1 + r1).astype(jnp.int32)

    tstart = (lax.broadcasted_iota(jnp.int32, (_E, _NTILES), 1)
              .astype(jnp.float32) * float(_M))
    eid = jnp.sum((tstart >= ends).astype(jnp.float32), axis=0, keepdims=True)
    eid = jnp.minimum(eid, float(_E - 1))
    eid_ref[...] = eid.astype(jnp.int32)

    ee1 = e1.astype(jnp.float32)
    ee2 = e2.astype(jnp.float32)
    pad_cols = lax.dot_general(jnp.ones((_E, 1), jnp.float32), pad,
                               (((1,), (1,)), ((), ())),
                               preferred_element_type=jnp.float32)
    cand = jnp.where((ee2 > ee1) * (pad_cols > 0).astype(jnp.float32) > 0,
                     ee2, 1e9)
    nxt = jnp.min(cand, axis=1, keepdims=True)
    nxt = jnp.where(nxt > float(_E), ee1[:, :1], nxt)
    eiota_nt = (lax.broadcasted_iota(jnp.int32, (_E, _NTILES), 0)
                .astype(jnp.float32))
    sel = (eiota_nt == eid).astype(jnp.float32)
    pnx_ref[...] = jnp.sum(sel * nxt, axis=0, keepdims=True).astype(jnp.int32)

    pres_before = jnp.sum(
        jnp.where((ee2 < ee1) * (pad_cols > 0).astype(jnp.float32) > 0,
                  1.0, 0.0), axis=1, keepdims=True)
    seg_ref[...] = jnp.sum(sel * pres_before, axis=0,
                           keepdims=True).astype(jnp.int32)

    f = counts / (float(_T * _K) + 1e-9)
    pmean = jnp.mean(scores, axis=1, keepdims=True)
    aux_ref[...] = jnp.sum(f * pmean, axis=0, keepdims=True) * (_COEFF * _E)


def _route(x2d, gate_w):
    return pl.pallas_call(
        _route_body,
        out_shape=(
            jax.ShapeDtypeStruct((1, _T), jnp.int32),
            jax.ShapeDtypeStruct((1, _T), jnp.int32),
            jax.ShapeDtypeStruct((1, _T), jnp.float32),
            jax.ShapeDtypeStruct((1, _T), jnp.float32),
            jax.ShapeDtypeStruct((1, _NTILES), jnp.int32),
            jax.ShapeDtypeStruct((1, _NTILES), jnp.int32),
            jax.ShapeDtypeStruct((1, _NTILES), jnp.int32),
            jax.ShapeDtypeStruct((1, 1), jnp.float32),
        ),
    )(x2d, gate_w)


def _build_dispatch(pos0, pos1, w0, w1):
    mesh = plsc.VectorSubcoreMesh(core_axis_name="c", subcore_axis_name="s")

    @functools.partial(
        pl.kernel, mesh=mesh,
        compiler_params=pltpu.CompilerParams(needs_layout_passes=False),
        out_type=(
            jax.ShapeDtypeStruct((_R,), jnp.int32),
            jax.ShapeDtypeStruct((_R,), jnp.float32),
        ),
        scratch_types=[
            pltpu.VMEM((_T,), jnp.int32),
            pltpu.VMEM((_T,), jnp.int32),
            pltpu.VMEM((_T,), jnp.float32),
            pltpu.VMEM((_T,), jnp.float32),
            pltpu.VMEM((_R,), jnp.int32),
            pltpu.VMEM((_R,), jnp.float32),
        ],
    )
    def k(p0_hbm, p1_hbm, w0_hbm, w1_hbm, idx_hbm, wrow_hbm,
          p0_v, p1_v, w0_v, w1_v, idx_v, wrow_v):
        wid = lax.axis_index("s") * 2 + lax.axis_index("c")

        @pl.when(wid == 0)
        def _():
            pltpu.sync_copy(p0_hbm, p0_v)
            pltpu.sync_copy(p1_hbm, p1_v)
            pltpu.sync_copy(w0_hbm, w0_v)
            pltpu.sync_copy(w1_hbm, w1_v)

            def zero(i, _):
                idx_v[pl.ds(i * 16, 16)] = jnp.zeros((16,), jnp.int32)
                wrow_v[pl.ds(i * 16, 16)] = jnp.zeros((16,), jnp.float32)
                return 0
            lax.fori_loop(0, _R // 16, zero, 0)

            def scat(i, _):
                tok = lax.iota(jnp.int32, 16) + i * 16
                sl = pl.ds(i * 16, 16)
                plsc.store_scatter(idx_v, [p0_v[sl]], tok)
                plsc.store_scatter(idx_v, [p1_v[sl]], tok)
                plsc.store_scatter(wrow_v, [p0_v[sl]], w0_v[sl])
                plsc.store_scatter(wrow_v, [p1_v[sl]], w1_v[sl])
                return 0
            lax.fori_loop(0, _T // 16, scat, 0)

            pltpu.sync_copy(idx_v, idx_hbm)
            pltpu.sync_copy(wrow_v, wrow_hbm)

    return k(pos0, pos1, w0, w1)


def _ffn_gather_body(eid_ref, pnx_ref, seg_ref, idx_ref, x_ref,
                     wg_ref, wu_ref, wd_ref, wrow_ref, out_ref,
                     xbuf, wgs, wus, wds, gsem, usem, dsem):
    i = pl.program_id(0)
    e = eid_ref[i]
    slot = lax.rem(seg_ref[i], 2)

    def issue(g, s):
        pltpu.make_async_copy(wg_ref.at[g], wgs.at[s], gsem.at[s]).start()
        pltpu.make_async_copy(wu_ref.at[g], wus.at[s], usem.at[s]).start()
        pltpu.make_async_copy(wd_ref.at[g], wds.at[s], dsem.at[s]).start()

    def wait_for(g, s):
        pltpu.make_async_copy(wg_ref.at[g], wgs.at[s], gsem.at[s]).wait()
        pltpu.make_async_copy(wu_ref.at[g], wus.at[s], usem.at[s]).wait()
        pltpu.make_async_copy(wd_ref.at[g], wds.at[s], dsem.at[s]).wait()

    @pl.when(i == 0)
    def _():
        issue(e, slot)

    boundary = (i == 0) | (eid_ref[jnp.maximum(i - 1, 0)] != e)

    @pl.when(boundary)
    def _():
        wait_for(e, slot)
        nx = pnx_ref[i]
        @pl.when(nx != e)
        def _():
            issue(nx, 1 - slot)

    wr = wrow_ref[...]
    @pl.when(jnp.max(jnp.abs(wr)) > 0)
    def _():
        for r in range(_M):
            t = idx_ref[i * _M + r]
            xbuf[pl.ds(r, 1), :] = x_ref[pl.ds(t, 1), :]
        xt = xbuf[...]
        g = lax.dot_general(xt, wgs[slot], (((1,), (1,)), ((), ())),
                            preferred_element_type=jnp.float32)
        u = lax.dot_general(xt, wus[slot], (((1,), (1,)), ((), ())),
                            preferred_element_type=jnp.float32)
        h1 = (g * jax.nn.sigmoid(g)) * u
        o = lax.dot_general(h1, wds[slot], (((1,), (1,)), ((), ())),
                            preferred_element_type=jnp.float32)
        out_ref[...] = o * wr


def _ffn_gather(x2d, idx, wg, wu, wd, wrow, eid, pnx, seg, ntiles):
    grid_spec = pltpu.PrefetchScalarGridSpec(
        num_scalar_prefetch=4,
        grid=(ntiles,),
        in_specs=[
            pl.BlockSpec((_T, _D), lambda i, e, p, s, ix: (0, 0)),
            pl.BlockSpec(memory_space=pl.ANY),
            pl.BlockSpec(memory_space=pl.ANY),
            pl.BlockSpec(memory_space=pl.ANY),
            pl.BlockSpec((_M, 1), lambda i, e, p, s, ix: (i, 0)),
        ],
        out_specs=pl.BlockSpec((_M, _D), lambda i, e, p, s, ix: (i, 0)),
        scratch_shapes=[
            pltpu.VMEM((_M, _D), jnp.float32),
            pltpu.VMEM((2, _F, _D), jnp.float32),
            pltpu.VMEM((2, _F, _D), jnp.float32),
            pltpu.VMEM((2, _D, _F), jnp.float32),
            pltpu.SemaphoreType.DMA((2,)),
            pltpu.SemaphoreType.DMA((2,)),
            pltpu.SemaphoreType.DMA((2,)),
        ],
    )
    return pl.pallas_call(
        _ffn_gather_body,
        grid_spec=grid_spec,
        out_shape=jax.ShapeDtypeStruct((ntiles * _M, _D), jnp.float32),
    )(eid, pnx, seg, idx, x2d, wg, wu, wd, wrow)


def _ffn_body(eid_ref, xs_ref, wg_ref, wu_ref, wd_ref, wrow_ref, out_ref):
    xt = xs_ref[...]
    g = lax.dot_general(xt, wg_ref[0], (((1,), (1,)), ((), ())),
                        preferred_element_type=jnp.float32)
    u = lax.dot_general(xt, wu_ref[0], (((1,), (1,)), ((), ())),
                        preferred_element_type=jnp.float32)
    h1 = (g * jax.nn.sigmoid(g)) * u
    o = lax.dot_general(h1, wd_ref[0], (((1,), (1,)), ((), ())),
                        preferred_element_type=jnp.float32)
    out_ref[...] = o * wrow_ref[...]


def _ffn(xs, wg, wu, wd, wrow, eid, ntiles):
    grid_spec = pltpu.PrefetchScalarGridSpec(
        num_scalar_prefetch=1,
        grid=(ntiles,),
        in_specs=[
            pl.BlockSpec((_M, _D), lambda i, eid_ref: (i, 0)),
            pl.BlockSpec((1, _F, _D), lambda i, eid_ref: (eid_ref[i], 0, 0)),
            pl.BlockSpec((1, _F, _D), lambda i, eid_ref: (eid_ref[i], 0, 0)),
            pl.BlockSpec((1, _D, _F), lambda i, eid_ref: (eid_ref[i], 0, 0)),
            pl.BlockSpec((_M, 1), lambda i, eid_ref: (i, 0)),
        ],
        out_specs=pl.BlockSpec((_M, _D), lambda i, eid_ref: (i, 0)),
    )
    return pl.pallas_call(
        _ffn_body,
        grid_spec=grid_spec,
        out_shape=jax.ShapeDtypeStruct((ntiles * _M, _D), jnp.float32),
    )(eid, xs, wg, wu, wd, wrow)


def _combine(ys_r, ys_s, pos0, pos1):
    mesh = plsc.VectorSubcoreMesh(core_axis_name="c", subcore_axis_name="s")
    tok_per_w = _T // _NW
    chunk = 16
    nch = tok_per_w // chunk

    nbuf = 2

    @functools.partial(
        pl.kernel, mesh=mesh,
        compiler_params=pltpu.CompilerParams(needs_layout_passes=False),
        out_type=jax.ShapeDtypeStruct((_T, _D), jnp.float32),
        scratch_types=[
            pltpu.VMEM((tok_per_w,), jnp.int32),
            pltpu.VMEM((tok_per_w,), jnp.int32),
            *([pltpu.VMEM((chunk, _D), jnp.float32)] * (3 * nbuf)),
            *([pltpu.SemaphoreType.DMA] * (3 * nbuf)),
            *([pltpu.SemaphoreType.DMA] * nbuf),
        ],
    )
    def k(ysr_hbm, yss_hbm, p0_hbm, p1_hbm, out_hbm, i0_v, i1_v, *scr):
        r0 = scr[0:nbuf]
        r1 = scr[nbuf:2 * nbuf]
        rs = scr[2 * nbuf:3 * nbuf]
        gsem = scr[3 * nbuf:6 * nbuf]
        ssem = scr[6 * nbuf:]
        wid = lax.axis_index("s") * 2 + lax.axis_index("c")
        base = wid * tok_per_w
        pltpu.sync_copy(p0_hbm.at[pl.ds(base, tok_per_w)], i0_v)
        pltpu.sync_copy(p1_hbm.at[pl.ds(base, tok_per_w)], i1_v)

        def start_loads(i, b):
            sl = pl.ds(i * chunk, chunk)
            return (
                pltpu.async_copy(ysr_hbm.at[i0_v.at[sl]], r0[b], gsem[b]),
                pltpu.async_copy(ysr_hbm.at[i1_v.at[sl]], r1[b], gsem[nbuf + b]),
                pltpu.async_copy(yss_hbm.at[pl.ds(base + i * chunk, chunk)],
                                 rs[b], gsem[2 * nbuf + b]),
            )

        gh = [None] * nbuf
        sh = [None] * nbuf
        waited = [True] * nbuf
        for i in range(min(nbuf, nch)):
            gh[i] = start_loads(i, i)
        for i in range(nch):
            b = i % nbuf
            for h in gh[b]:
                h.wait()
            for r in range(chunk):
                def add(cc, _):
                    sl = pl.ds(cc * 16, 16)
                    rs[b][r, sl] = rs[b][r, sl] + r0[b][r, sl] + r1[b][r, sl]
                    return 0
                lax.fori_loop(0, _D // 16, add, 0, unroll=8)
            sh[b] = pltpu.async_copy(
                rs[b], out_hbm.at[pl.ds(base + i * chunk, chunk)], ssem[b])
            waited[b] = False
            if i + nbuf < nch:
                sh[b].wait()
                waited[b] = True
                gh[b] = start_loads(i + nbuf, b)
        for b in range(nbuf):
            if not waited[b]:
                sh[b].wait()

    return k(ys_r, ys_s, pos0, pos1)


def kernel(x, shared_gate, shared_up, shared_down,
           routed_gate, routed_up, routed_down, gate_w):
    Bx, Sx, Dx = x.shape
    x2d = x.reshape(_T, _D)

    pos0, pos1, w0, w1, eid, pnx, seg, aux = _route(x2d, gate_w)
    pos0 = pos0.reshape(_T)
    pos1 = pos1.reshape(_T)
    w0 = w0.reshape(_T)
    w1 = w1.reshape(_T)
    eid = eid.reshape(_NTILES)
    pnx = pnx.reshape(_NTILES)
    seg = seg.reshape(_NTILES)

    ones = jnp.ones((_T, 1), jnp.float32)
    eid_s = jnp.zeros((_T // _M,), jnp.int32)
    ys_s = _ffn(x2d, shared_gate, shared_up, shared_down,
                ones, eid_s, _T // _M)

    idx, wrow = _build_dispatch(pos0, pos1, w0, w1)

    ys_r = _ffn_gather(x2d, idx, routed_gate, routed_up, routed_down,
                       wrow.reshape(_R, 1), eid, pnx, seg, _NTILES)

    out2d = _combine(ys_r, ys_s, pos0, pos1)
    return out2d.reshape(Bx, Sx, Dx), aux[0, 0]
